# Initial kernel scaffold; baseline (speedup 1.0000x reference)
import jax
import jax.numpy as jnp
from jax.experimental import pallas as pl


def kernel(tseq, grid, deform_table, dW0, dW1, dW2, video_table, vW0, vW1, vW2):
    raise NotImplementedError("write your pallas kernel here")


# speedup vs baseline: 2.7562x; 2.7562x over previous
import functools

import numpy as np
import jax
import jax.numpy as jnp
from jax import lax
from jax.experimental import pallas as pl
from jax.experimental.pallas import tpu as pltpu
from jax.experimental.pallas import tpu_sc as plsc

_N_LEVELS = 16
_T = 1 << 19
_MASK = np.int32(_T - 1)
_P1 = np.int32(-1640531535)
_P2 = np.int32(805459861)
_RES = [int(np.floor(16 * (1.5 ** l))) for l in range(_N_LEVELS)]
_NC, _NS = 2, 16
_NW = _NC * _NS
_N = 262144
_PPW = _N // _NW


def _encode_call(coords_flat, table_flat, tpari, tparf, three_d):
    NT = 2 if three_d else 1
    C = 16
    R = _N_LEVELS * NT
    NCH = _PPW // C
    D = 3 if three_d else 2
    dense = [(r + 1) ** D <= _T for r in _RES]
    mesh = plsc.VectorSubcoreMesh(core_axis_name="c", subcore_axis_name="s")

    scratch = [
        pltpu.VMEM((2 * _PPW,), jnp.float32),
        pltpu.VMEM((R, 128), jnp.int32),
        pltpu.VMEM((R, 64), jnp.float32),
        pltpu.VMEM((R, 128), jnp.float32),
        pltpu.VMEM((C, 32), jnp.float32),
        pltpu.SemaphoreType.DMA,
    ]
    if three_d:
        scratch += [pltpu.VMEM((2 * _N_LEVELS, 16), jnp.int32),
                    pltpu.VMEM((2 * _N_LEVELS, 16), jnp.float32)]

    def body(*args):
        if three_d:
            (coords_hbm, tpari_hbm, tparf_hbm, table_hbm, out_hbm,
             cv, idxv, wv, rowsv, encv, sem, tpiv, tpfv) = args
        else:
            (coords_hbm, table_hbm, out_hbm,
             cv, idxv, wv, rowsv, encv, sem) = args
        wid = lax.axis_index("s") * _NC + lax.axis_index("c")
        base = wid * _PPW
        pltpu.sync_copy(coords_hbm.at[pl.ds(base * 2, _PPW * 2)], cv)
        if three_d:
            pltpu.sync_copy(tpari_hbm, tpiv)
            pltpu.sync_copy(tparf_hbm, tpfv)

        def chunk(ci, carry):
            cb = ci * C
            coff = (cb // 128) * 256 + (cb % 128)
            xg = cv[pl.ds(coff, 16)]
            yg = cv[pl.ds(coff + 128, 16)]
            for l in range(_N_LEVELS):
                res = _RES[l]
                s = res + 1
                L20 = l << 20
                px = xg * res
                py = yg * res
                ix = jnp.clip(px.astype(jnp.int32), 0, res - 1)
                iy = jnp.clip(py.astype(jnp.int32), 0, res - 1)
                fx = px - ix.astype(jnp.float32)
                fy = py - iy.astype(jnp.float32)
                wx0 = 1.0 - fx
                wy0 = 1.0 - fy
                w4 = (wx0 * wy0, fx * wy0, wx0 * fy, fx * fy)
                if dense[l]:
                    b00 = ix + iy * s
                    cidx = (b00, b00 + 1, b00 + s, b00 + s + 1)
                else:
                    hy0 = iy * _P1
                    hy1 = hy0 + _P1
                    cidx = (ix ^ hy0, (ix + 1) ^ hy0,
                            ix ^ hy1, (ix + 1) ^ hy1)
                for tc in range(NT):
                    r = l * NT + tc
                    if three_d:
                        ct = tpiv[2 * l + tc, :]
                        wt = tpfv[2 * l + tc, :]
                    for c in range(4):
                        if three_d:
                            if dense[l]:
                                cell = cidx[c] + ct
                            else:
                                cell = (cidx[c] ^ ct) & _MASK
                            wc = w4[c] * wt
                        else:
                            if dense[l]:
                                cell = cidx[c]
                            else:
                                cell = cidx[c] & _MASK
                            wc = w4[c]
                        ph = (cell + lax.shift_left(
                            lax.shift_right_logical(cell, 7), 7)) + L20
                        idxv[r, pl.ds(c * 16, 16)] = ph
                        idxv[r, pl.ds(64 + c * 16, 16)] = ph + 128
                        wv[r, pl.ds(c * 16, 16)] = wc
            cps = [pltpu.async_copy(table_hbm.at[idxv.at[r]], rowsv.at[r], sem)
                   for r in range(R)]
            for cp in cps:
                cp.wait()
            lanes = lax.iota(jnp.int32, 16)
            for l in range(_N_LEVELS):
                wrows = [[wv[l * NT + tc, pl.ds(c * 16, 16)]
                          for c in range(4)] for tc in range(NT)]
                for f in range(2):
                    acc = None
                    for tc in range(NT):
                        r = l * NT + tc
                        for c in range(4):
                            vals = rowsv[r, pl.ds(f * 64 + c * 16, 16)]
                            term = vals * wrows[tc][c]
                            acc = term if acc is None else acc + term
                    plsc.store_scatter(
                        encv, [lanes, jnp.full((16,), 2 * l + f, jnp.int32)],
                        acc)
            pltpu.sync_copy(encv, out_hbm.at[pl.ds(base + cb, C)])
            return carry

        lax.fori_loop(0, NCH, chunk, 0)

    kern = pl.kernel(body,
                     out_type=jax.ShapeDtypeStruct((_N, 32), jnp.float32),
                     mesh=mesh, scratch_types=scratch,
                     compiler_params=pltpu.CompilerParams(
                         needs_layout_passes=False))
    if three_d:
        return kern(coords_flat, tpari, tparf, table_flat)
    return kern(coords_flat, table_flat)


def _mlp_call(pe_blocks_in, f2, exrow, enc, w0c, w0e, w1, w2, n_out,
              is_deform):
    BLK = 2048
    grid_steps = _N // BLK
    BR = BLK // 128

    def body(fin_ref, ex_ref, enc_ref, w0c_ref, w0e_ref, w1_ref, w2_ref,
             out_ref):
        fin = fin_ref[...]
        f2b = fin.reshape(BR, 2, 128).swapaxes(1, 2).reshape(BLK, 2)
        h = jnp.dot(enc_ref[...], w0e_ref[...],
                    preferred_element_type=jnp.float32)
        h = h + jnp.dot(f2b, w0c_ref[...],
                        preferred_element_type=jnp.float32)
        h = h + ex_ref[...]
        h = jnp.maximum(h, 0.0)
        h = jnp.maximum(jnp.dot(h, w1_ref[...],
                                preferred_element_type=jnp.float32), 0.0)
        o = jnp.dot(h, w2_ref[...], preferred_element_type=jnp.float32)
        if is_deform:
            pe = (o / 5.0 + f2b + 0.3) / 1.6
            out_ref[...] = pe.reshape(BR, 128, 2).swapaxes(1, 2).reshape(
                2 * BR, 128)
        else:
            out_ref[...] = o

    if is_deform:
        out_shape = jax.ShapeDtypeStruct((_N // 128 * 2, 128), jnp.float32)
        out_spec = pl.BlockSpec((2 * BR, 128), lambda i: (i, 0))
    else:
        out_shape = jax.ShapeDtypeStruct((_N, n_out), jnp.float32)
        out_spec = pl.BlockSpec((BLK, n_out), lambda i: (i, 0))

    fin = pe_blocks_in if pe_blocks_in is not None else f2
    return pl.pallas_call(
        body,
        grid=(grid_steps,),
        in_specs=[
            pl.BlockSpec((2 * BR, 128), lambda i: (i, 0)),
            pl.BlockSpec((1, 64), lambda i: (0, 0)),
            pl.BlockSpec((BLK, 32), lambda i: (i, 0)),
            pl.BlockSpec((2, 64), lambda i: (0, 0)),
            pl.BlockSpec((32, 64), lambda i: (0, 0)),
            pl.BlockSpec((64, 64), lambda i: (0, 0)),
            pl.BlockSpec((64, n_out), lambda i: (0, 0)),
        ],
        out_specs=out_spec,
        out_shape=out_shape,
    )(fin, exrow, enc, w0c, w0e, w1, w2)


def kernel(tseq, grid, deform_table, dW0, dW1, dW2, video_table, vW0, vW1,
           vW2):
    gblocks = grid.reshape(_N // 128, 128, 2).swapaxes(1, 2).reshape(
        _N // 128 * 2, 128)
    gflat = gblocks.reshape(-1)
    dtab = deform_table.reshape(_N_LEVELS, _T // 128, 128, 2).swapaxes(
        2, 3).reshape(-1)
    vtab = video_table.reshape(_N_LEVELS, _T // 128, 128, 2).swapaxes(
        2, 3).reshape(-1)

    t = tseq[0, 0]
    cti, ctf = [], []
    for l in range(_N_LEVELS):
        res = _RES[l]
        s = res + 1
        pt = t * res
        it0f = jnp.clip(jnp.floor(pt), 0.0, float(res - 1))
        ft = pt - it0f
        it0 = it0f.astype(jnp.int32)
        it1 = it0 + 1
        if s ** 3 <= _T:
            ct0 = it0 * (s * s)
            ct1 = it1 * (s * s)
        else:
            ct0 = it0 * _P2
            ct1 = it1 * _P2
        cti += [ct0, ct1]
        ctf += [1.0 - ft, ft]
    tpari = jnp.broadcast_to(jnp.stack(cti)[:, None],
                             (2 * _N_LEVELS, 16)).astype(jnp.int32)
    tparf = jnp.broadcast_to(jnp.stack(ctf)[:, None],
                             (2 * _N_LEVELS, 16)).astype(jnp.float32)

    enc1 = _encode_call(gflat, dtab, tpari, tparf, three_d=True)
    exrow = t * dW0[2:3, :]
    pe_blocks = _mlp_call(None, gblocks, exrow, enc1, dW0[:2], dW0[3:],
                          dW1, dW2, 2, is_deform=True)
    enc2 = _encode_call(pe_blocks.reshape(-1), vtab, None, None,
                        three_d=False)
    out = _mlp_call(pe_blocks, None, jnp.zeros((1, 64), jnp.float32), enc2,
                    vW0[:2], vW0[2:], vW1, vW2, 3, is_deform=False)
    return out


# speedup vs baseline: 2.8800x; 1.0449x over previous
import functools

import numpy as np
import jax
import jax.numpy as jnp
from jax import lax
from jax.experimental import pallas as pl
from jax.experimental.pallas import tpu as pltpu
from jax.experimental.pallas import tpu_sc as plsc

_N_LEVELS = 16
_T = 1 << 19
_MASK = np.int32(_T - 1)
_P1 = np.int32(-1640531535)
_P2 = np.int32(805459861)
_RES = [int(np.floor(16 * (1.5 ** l))) for l in range(_N_LEVELS)]
_NC, _NS = 2, 16
_NW = _NC * _NS
_N = 262144
_PPW = _N // _NW


def _encode_call(coords_flat, table_flat, tpari, tparf, three_d):
    NT = 2 if three_d else 1
    C = 16
    R = _N_LEVELS * NT
    NCH = _PPW // C
    D = 3 if three_d else 2
    dense = [(r + 1) ** D <= _T for r in _RES]
    mesh = plsc.VectorSubcoreMesh(core_axis_name="c", subcore_axis_name="s")

    scratch = [
        pltpu.VMEM((2 * _PPW,), jnp.float32),
        pltpu.VMEM((2, R, 128), jnp.int32),
        pltpu.VMEM((2, R, 64), jnp.float32),
        pltpu.VMEM((2, R * 128), jnp.float32),
        pltpu.VMEM((2, C, 32), jnp.float32),
        pltpu.SemaphoreType.DMA,
        pltpu.SemaphoreType.DMA,
        pltpu.SemaphoreType.DMA,
        pltpu.SemaphoreType.DMA,
    ]
    if three_d:
        scratch += [pltpu.VMEM((2 * _N_LEVELS, 16), jnp.int32),
                    pltpu.VMEM((2 * _N_LEVELS, 16), jnp.float32)]

    def body(*args):
        if three_d:
            (coords_hbm, tpari_hbm, tparf_hbm, table_hbm, out_hbm,
             cv, idxv, wv, rowsv, encv, gsem0, gsem1, osem0, osem1,
             tpiv, tpfv) = args
        else:
            (coords_hbm, table_hbm, out_hbm,
             cv, idxv, wv, rowsv, encv, gsem0, gsem1, osem0, osem1) = args
        gsems = (gsem0, gsem1)
        osems = (osem0, osem1)
        wid = lax.axis_index("s") * _NC + lax.axis_index("c")
        base = wid * _PPW
        pltpu.sync_copy(coords_hbm.at[pl.ds(base * 2, _PPW * 2)], cv)
        if three_d:
            pltpu.sync_copy(tpari_hbm, tpiv)
            pltpu.sync_copy(tparf_hbm, tpfv)

        def gen_fire(ci, b):
            cb = ci * C
            coff = (cb // 128) * 256 + (cb % 128)
            xg = cv[pl.ds(coff, 16)]
            yg = cv[pl.ds(coff + 128, 16)]
            for l in range(_N_LEVELS):
                res = _RES[l]
                s = res + 1
                L20 = l << 20
                px = xg * res
                py = yg * res
                ix = jnp.clip(px.astype(jnp.int32), 0, res - 1)
                iy = jnp.clip(py.astype(jnp.int32), 0, res - 1)
                fx = px - ix.astype(jnp.float32)
                fy = py - iy.astype(jnp.float32)
                wx0 = 1.0 - fx
                wy0 = 1.0 - fy
                w4 = (wx0 * wy0, fx * wy0, wx0 * fy, fx * fy)
                if dense[l]:
                    b00 = ix + iy * s
                    cidx = (b00, b00 + 1, b00 + s, b00 + s + 1)
                else:
                    hy0 = iy * _P1
                    hy1 = hy0 + _P1
                    cidx = (ix ^ hy0, (ix + 1) ^ hy0,
                            ix ^ hy1, (ix + 1) ^ hy1)
                for tc in range(NT):
                    r = l * NT + tc
                    if three_d:
                        ct = tpiv[2 * l + tc, :]
                        wt = tpfv[2 * l + tc, :]
                    for c in range(4):
                        if three_d:
                            if dense[l]:
                                cell = cidx[c] + ct
                            else:
                                cell = (cidx[c] ^ ct) & _MASK
                            wc = w4[c] * wt
                        else:
                            if dense[l]:
                                cell = cidx[c]
                            else:
                                cell = cidx[c] & _MASK
                            wc = w4[c]
                        ph = (cell + lax.shift_left(
                            lax.shift_right_logical(cell, 7), 7)) + L20
                        idxv[b, r, pl.ds(c * 16, 16)] = ph
                        idxv[b, r, pl.ds(64 + c * 16, 16)] = ph + 128
                        wv[b, r, pl.ds(c * 16, 16)] = wc
            for r in range(R):
                pltpu.async_copy(table_hbm.at[idxv.at[b, r]],
                                 rowsv.at[b, pl.ds(r * 128, 128)], gsems[b])

        def wait_gathers(b):
            for r in range(R):
                pltpu.make_async_copy(
                    table_hbm.at[idxv.at[b, r]],
                    rowsv.at[b, pl.ds(r * 128, 128)], gsems[b]).wait()

        def drain_out(b):
            pltpu.make_async_copy(
                encv.at[b], out_hbm.at[pl.ds(base, C)], osems[b]).wait()

        def accum_fire_out(ci, b):
            cb = ci * C
            lanes = lax.iota(jnp.int32, 16)
            for l in range(_N_LEVELS):
                wrows = [[wv[b, l * NT + tc, pl.ds(c * 16, 16)]
                          for c in range(4)] for tc in range(NT)]
                for f in range(2):
                    acc = None
                    for tc in range(NT):
                        rb = (l * NT + tc) * 128
                        for c in range(4):
                            vals = rowsv[b, pl.ds(rb + f * 64 + c * 16, 16)]
                            term = vals * wrows[tc][c]
                            acc = term if acc is None else acc + term
                    plsc.store_scatter(
                        encv.at[b],
                        [lanes, jnp.full((16,), 2 * l + f, jnp.int32)], acc)
            pltpu.async_copy(encv.at[b], out_hbm.at[pl.ds(base + cb, C)],
                             osems[b])

        gen_fire(0, 0)

        def pair(i2, carry):
            i = i2 * 2
            gen_fire(i + 1, 1)
            wait_gathers(0)
            pl.when(i2 > 0)(lambda: drain_out(0))
            accum_fire_out(i, 0)
            pl.when(i2 < NCH // 2 - 1)(lambda: gen_fire(i + 2, 0))
            wait_gathers(1)
            pl.when(i2 > 0)(lambda: drain_out(1))
            accum_fire_out(i + 1, 1)
            return carry

        lax.fori_loop(0, NCH // 2, pair, 0)
        drain_out(0)
        drain_out(1)

    kern = pl.kernel(body,
                     out_type=jax.ShapeDtypeStruct((_N, 32), jnp.float32),
                     mesh=mesh, scratch_types=scratch,
                     compiler_params=pltpu.CompilerParams(
                         needs_layout_passes=False))
    if three_d:
        return kern(coords_flat, tpari, tparf, table_flat)
    return kern(coords_flat, table_flat)


def _mlp_call(pe_blocks_in, f2, exrow, enc, w0c, w0e, w1, w2, n_out,
              is_deform):
    BLK = 2048
    grid_steps = _N // BLK
    BR = BLK // 128

    def body(fin_ref, ex_ref, enc_ref, w0c_ref, w0e_ref, w1_ref, w2_ref,
             out_ref):
        fin = fin_ref[...]
        f2b = fin.reshape(BR, 2, 128).swapaxes(1, 2).reshape(BLK, 2)
        h = jnp.dot(enc_ref[...], w0e_ref[...],
                    preferred_element_type=jnp.float32)
        h = h + jnp.dot(f2b, w0c_ref[...],
                        preferred_element_type=jnp.float32)
        h = h + ex_ref[...]
        h = jnp.maximum(h, 0.0)
        h = jnp.maximum(jnp.dot(h, w1_ref[...],
                                preferred_element_type=jnp.float32), 0.0)
        o = jnp.dot(h, w2_ref[...], preferred_element_type=jnp.float32)
        if is_deform:
            pe = (o / 5.0 + f2b + 0.3) / 1.6
            out_ref[...] = pe.reshape(BR, 128, 2).swapaxes(1, 2).reshape(
                2 * BR, 128)
        else:
            out_ref[...] = o

    if is_deform:
        out_shape = jax.ShapeDtypeStruct((_N // 128 * 2, 128), jnp.float32)
        out_spec = pl.BlockSpec((2 * BR, 128), lambda i: (i, 0))
    else:
        out_shape = jax.ShapeDtypeStruct((_N, n_out), jnp.float32)
        out_spec = pl.BlockSpec((BLK, n_out), lambda i: (i, 0))

    fin = pe_blocks_in if pe_blocks_in is not None else f2
    return pl.pallas_call(
        body,
        grid=(grid_steps,),
        in_specs=[
            pl.BlockSpec((2 * BR, 128), lambda i: (i, 0)),
            pl.BlockSpec((1, 64), lambda i: (0, 0)),
            pl.BlockSpec((BLK, 32), lambda i: (i, 0)),
            pl.BlockSpec((2, 64), lambda i: (0, 0)),
            pl.BlockSpec((32, 64), lambda i: (0, 0)),
            pl.BlockSpec((64, 64), lambda i: (0, 0)),
            pl.BlockSpec((64, n_out), lambda i: (0, 0)),
        ],
        out_specs=out_spec,
        out_shape=out_shape,
    )(fin, exrow, enc, w0c, w0e, w1, w2)


def kernel(tseq, grid, deform_table, dW0, dW1, dW2, video_table, vW0, vW1,
           vW2):
    gblocks = grid.reshape(_N // 128, 128, 2).swapaxes(1, 2).reshape(
        _N // 128 * 2, 128)
    gflat = gblocks.reshape(-1)
    dtab = deform_table.reshape(_N_LEVELS, _T // 128, 128, 2).swapaxes(
        2, 3).reshape(-1)
    vtab = video_table.reshape(_N_LEVELS, _T // 128, 128, 2).swapaxes(
        2, 3).reshape(-1)

    t = tseq[0, 0]
    cti, ctf = [], []
    for l in range(_N_LEVELS):
        res = _RES[l]
        s = res + 1
        pt = t * res
        it0f = jnp.clip(jnp.floor(pt), 0.0, float(res - 1))
        ft = pt - it0f
        it0 = it0f.astype(jnp.int32)
        it1 = it0 + 1
        if s ** 3 <= _T:
            ct0 = it0 * (s * s)
            ct1 = it1 * (s * s)
        else:
            ct0 = it0 * _P2
            ct1 = it1 * _P2
        cti += [ct0, ct1]
        ctf += [1.0 - ft, ft]
    tpari = jnp.broadcast_to(jnp.stack(cti)[:, None],
                             (2 * _N_LEVELS, 16)).astype(jnp.int32)
    tparf = jnp.broadcast_to(jnp.stack(ctf)[:, None],
                             (2 * _N_LEVELS, 16)).astype(jnp.float32)

    enc1 = _encode_call(gflat, dtab, tpari, tparf, three_d=True)
    exrow = t * dW0[2:3, :]
    pe_blocks = _mlp_call(None, gblocks, exrow, enc1, dW0[:2], dW0[3:],
                          dW1, dW2, 2, is_deform=True)
    enc2 = _encode_call(pe_blocks.reshape(-1), vtab, None, None,
                        three_d=False)
    out = _mlp_call(pe_blocks, None, jnp.zeros((1, 64), jnp.float32), enc2,
                    vW0[:2], vW0[2:], vW1, vW2, 3, is_deform=False)
    return out


# speedup vs baseline: 2.9248x; 1.0156x over previous
import functools

import numpy as np
import jax
import jax.numpy as jnp
from jax import lax
from jax.experimental import pallas as pl
from jax.experimental.pallas import tpu as pltpu
from jax.experimental.pallas import tpu_sc as plsc

_N_LEVELS = 16
_T = 1 << 19
_MASK = np.int32(_T - 1)
_P1 = np.int32(-1640531535)
_P2 = np.int32(805459861)
_RES = [int(np.floor(16 * (1.5 ** l))) for l in range(_N_LEVELS)]
_NC, _NS = 2, 16
_NW = _NC * _NS
_N = 262144
_PPW = _N // _NW


def _encode_call(coords_flat, table_flat, tpari, tparf, three_d):
    NT = 2 if three_d else 1
    LEVELS = list(range(_N_LEVELS))
    C = 16
    R = len(LEVELS) * NT
    NCH = _PPW // C
    D = 3 if three_d else 2
    dense = [(r + 1) ** D <= _T for r in _RES]
    mesh = plsc.VectorSubcoreMesh(core_axis_name="c", subcore_axis_name="s")

    scratch = [
        pltpu.VMEM((2 * _PPW,), jnp.float32),
        pltpu.VMEM((2, R, 128), jnp.int32),
        pltpu.VMEM((2, R, 64), jnp.float32),
        pltpu.VMEM((2, R * 128), jnp.float32),
        pltpu.VMEM((32, 128), jnp.float32),
        pltpu.SemaphoreType.DMA,
        pltpu.SemaphoreType.DMA,
        pltpu.SemaphoreType.DMA,
    ]
    if three_d:
        scratch += [pltpu.VMEM((2 * _N_LEVELS, 16), jnp.int32),
                    pltpu.VMEM((2 * _N_LEVELS, 16), jnp.float32)]

    def body(*args):
        if three_d:
            (coords_hbm, tpari_hbm, tparf_hbm, table_hbm, out_hbm,
             cv, idxv, wv, rowsv, encv, gsem0, gsem1, osem,
             tpiv, tpfv) = args
        else:
            (coords_hbm, table_hbm, out_hbm,
             cv, idxv, wv, rowsv, encv, gsem0, gsem1, osem) = args
        gsems = (gsem0, gsem1)
        wid = lax.axis_index("s") * _NC + lax.axis_index("c")
        base = wid * _PPW
        pltpu.sync_copy(coords_hbm.at[pl.ds(base * 2, _PPW * 2)], cv)
        if three_d:
            pltpu.sync_copy(tpari_hbm, tpiv)
            pltpu.sync_copy(tparf_hbm, tpfv)

        def gen_fire(ci, b):
            cb = ci * C
            coff = (cb // 128) * 256 + (cb % 128)
            xg = cv[pl.ds(coff, 16)]
            yg = cv[pl.ds(coff + 128, 16)]
            for li, l in enumerate(LEVELS):
                res = _RES[l]
                s = res + 1
                L20 = l << 20
                px = xg * res
                py = yg * res
                ix = jnp.clip(px.astype(jnp.int32), 0, res - 1)
                iy = jnp.clip(py.astype(jnp.int32), 0, res - 1)
                fx = px - ix.astype(jnp.float32)
                fy = py - iy.astype(jnp.float32)
                wx0 = 1.0 - fx
                wy0 = 1.0 - fy
                w4 = (wx0 * wy0, fx * wy0, wx0 * fy, fx * fy)
                if dense[l]:
                    b00 = ix + iy * s
                    cidx = (b00, b00 + 1, b00 + s, b00 + s + 1)
                else:
                    hy0 = iy * _P1
                    hy1 = hy0 + _P1
                    cidx = (ix ^ hy0, (ix + 1) ^ hy0,
                            ix ^ hy1, (ix + 1) ^ hy1)
                for tc in range(NT):
                    r = li * NT + tc
                    if three_d:
                        ct = tpiv[2 * l + tc, :]
                        wt = tpfv[2 * l + tc, :]
                    for c in range(4):
                        if three_d:
                            if dense[l]:
                                cell = cidx[c] + ct
                            else:
                                cell = (cidx[c] ^ ct) & _MASK
                            wc = w4[c] * wt
                        else:
                            if dense[l]:
                                cell = cidx[c]
                            else:
                                cell = cidx[c] & _MASK
                            wc = w4[c]
                        ph = (cell + lax.shift_left(
                            lax.shift_right_logical(cell, 7), 7)) + L20
                        idxv[b, r, pl.ds(c * 16, 16)] = ph
                        idxv[b, r, pl.ds(64 + c * 16, 16)] = ph + 128
                        wv[b, r, pl.ds(c * 16, 16)] = wc
            for r in range(R):
                pltpu.async_copy(table_hbm.at[idxv.at[b, r]],
                                 rowsv.at[b, pl.ds(r * 128, 128)], gsems[b])

        def wait_gathers(b):
            for r in range(R):
                pltpu.make_async_copy(
                    table_hbm.at[idxv.at[b, r]],
                    rowsv.at[b, pl.ds(r * 128, 128)], gsems[b]).wait()

        bbase = wid * (_PPW // 128)

        def drain_out():
            pltpu.make_async_copy(
                encv, out_hbm.at[bbase], osem).wait()

        def accum(ci, b):
            cb = ci * C
            suboff = cb % 128
            for li, l in enumerate(LEVELS):
                wrows = [[wv[b, li * NT + tc, pl.ds(c * 16, 16)]
                          for c in range(4)] for tc in range(NT)]
                for f in range(2):
                    acc = None
                    for tc in range(NT):
                        rb = (li * NT + tc) * 128
                        for c in range(4):
                            vals = rowsv[b, pl.ds(rb + f * 64 + c * 16, 16)]
                            term = vals * wrows[tc][c]
                            acc = term if acc is None else acc + term
                    encv[2 * l + f, pl.ds(suboff, 16)] = acc

        def fire_out(ci):
            blk = bbase + ci // 8
            pltpu.async_copy(encv, out_hbm.at[blk], osem)

        gen_fire(0, 0)

        def pair(i2, carry):
            i = i2 * 2
            gen_fire(i + 1, 1)
            wait_gathers(0)
            pl.when(jnp.logical_and(i2 % 4 == 0, i2 > 0))(drain_out)
            accum(i, 0)
            pl.when(i2 < NCH // 2 - 1)(lambda: gen_fire(i + 2, 0))
            wait_gathers(1)
            accum(i + 1, 1)
            pl.when(i2 % 4 == 3)(lambda: fire_out(i + 1))
            return carry

        lax.fori_loop(0, NCH // 2, pair, 0)
        drain_out()

    kern = pl.kernel(body,
                     out_type=jax.ShapeDtypeStruct((_N // 128, 32, 128),
                                                   jnp.float32),
                     mesh=mesh, scratch_types=scratch,
                     compiler_params=pltpu.CompilerParams(
                         needs_layout_passes=False))
    if three_d:
        return kern(coords_flat, tpari, tparf, table_flat)
    return kern(coords_flat, table_flat)


def _mlp_call(pe_blocks_in, f2, exrow, enc, w0c, w0e, w1, w2, n_out,
              is_deform):
    BLK = 2048
    grid_steps = _N // BLK
    BR = BLK // 128

    def body(fin_ref, ex_ref, enc_ref, w0c_ref, w0e_ref, w1_ref, w2_ref,
             out_ref):
        fin = fin_ref[...]
        f2b = fin.reshape(BR, 2, 128).swapaxes(1, 2).reshape(BLK, 2)
        encb = enc_ref[...].swapaxes(1, 2).reshape(BLK, 32)
        h = jnp.dot(encb, w0e_ref[...],
                    preferred_element_type=jnp.float32)
        h = h + jnp.dot(f2b, w0c_ref[...],
                        preferred_element_type=jnp.float32)
        h = h + ex_ref[...]
        h = jnp.maximum(h, 0.0)
        h = jnp.maximum(jnp.dot(h, w1_ref[...],
                                preferred_element_type=jnp.float32), 0.0)
        o = jnp.dot(h, w2_ref[...], preferred_element_type=jnp.float32)
        if is_deform:
            pe = (o / 5.0 + f2b + 0.3) / 1.6
            out_ref[...] = pe.reshape(BR, 128, 2).swapaxes(1, 2).reshape(
                2 * BR, 128)
        else:
            out_ref[...] = o

    if is_deform:
        out_shape = jax.ShapeDtypeStruct((_N // 128 * 2, 128), jnp.float32)
        out_spec = pl.BlockSpec((2 * BR, 128), lambda i: (i, 0))
    else:
        out_shape = jax.ShapeDtypeStruct((_N, n_out), jnp.float32)
        out_spec = pl.BlockSpec((BLK, n_out), lambda i: (i, 0))

    fin = pe_blocks_in if pe_blocks_in is not None else f2
    return pl.pallas_call(
        body,
        grid=(grid_steps,),
        in_specs=[
            pl.BlockSpec((2 * BR, 128), lambda i: (i, 0)),
            pl.BlockSpec((1, 64), lambda i: (0, 0)),
            pl.BlockSpec((BR, 32, 128), lambda i: (i, 0, 0)),
            pl.BlockSpec((2, 64), lambda i: (0, 0)),
            pl.BlockSpec((32, 64), lambda i: (0, 0)),
            pl.BlockSpec((64, 64), lambda i: (0, 0)),
            pl.BlockSpec((64, n_out), lambda i: (0, 0)),
        ],
        out_specs=out_spec,
        out_shape=out_shape,
    )(fin, exrow, enc, w0c, w0e, w1, w2)


def kernel(tseq, grid, deform_table, dW0, dW1, dW2, video_table, vW0, vW1,
           vW2):
    gblocks = grid.reshape(_N // 128, 128, 2).swapaxes(1, 2).reshape(
        _N // 128 * 2, 128)
    gflat = gblocks.reshape(-1)
    dtab = deform_table.reshape(_N_LEVELS, _T // 128, 128, 2).swapaxes(
        2, 3).reshape(-1)
    vtab = video_table.reshape(_N_LEVELS, _T // 128, 128, 2).swapaxes(
        2, 3).reshape(-1)

    t = tseq[0, 0]
    cti, ctf = [], []
    for l in range(_N_LEVELS):
        res = _RES[l]
        s = res + 1
        pt = t * res
        it0f = jnp.clip(jnp.floor(pt), 0.0, float(res - 1))
        ft = pt - it0f
        it0 = it0f.astype(jnp.int32)
        it1 = it0 + 1
        if s ** 3 <= _T:
            ct0 = it0 * (s * s)
            ct1 = it1 * (s * s)
        else:
            ct0 = it0 * _P2
            ct1 = it1 * _P2
        cti += [ct0, ct1]
        ctf += [1.0 - ft, ft]
    tpari = jnp.broadcast_to(jnp.stack(cti)[:, None],
                             (2 * _N_LEVELS, 16)).astype(jnp.int32)
    tparf = jnp.broadcast_to(jnp.stack(ctf)[:, None],
                             (2 * _N_LEVELS, 16)).astype(jnp.float32)

    enc1 = _encode_call(gflat, dtab, tpari, tparf, three_d=True)
    exrow = t * dW0[2:3, :]
    pe_blocks = _mlp_call(None, gblocks, exrow, enc1, dW0[:2], dW0[3:],
                          dW1, dW2, 2, is_deform=True)
    enc2 = _encode_call(pe_blocks.reshape(-1), vtab, None, None,
                        three_d=False)
    out = _mlp_call(pe_blocks, None, jnp.zeros((1, 64), jnp.float32), enc2,
                    vW0[:2], vW0[2:], vW1, vW2, 3, is_deform=False)
    return out
